# baseline (speedup 1.0000x reference)
import functools

import jax
import jax.numpy as jnp
from jax import lax
from jax.experimental import pallas as pl
from jax.experimental.pallas import tpu as pltpu
from jax.experimental.pallas import tpu_sc as plsc

D = 1024
GROUP = 64
L = 16
NBUF = 4


def _sc_body(x_hbm, idx_hbm, msk_hbm, pe_hbm, out_hbm,
             idx_v, msk_v, perm_v, pev_v,
             buf0, buf1, buf2, buf3,
             sst0, sst1, sst2, sst3,
             tst0, tst1, tst2, tst3,
             sem0, sem1, sem2, sem3, *, rows_per_worker):
    c = rows_per_worker
    ng = c // L
    bufs = [buf0, buf1, buf2, buf3]
    ssts = [sst0, sst1, sst2, sst3]
    tsts = [tst0, tst1, tst2, tst3]
    sems = [sem0, sem1, sem2, sem3]

    nc = 2
    wid = lax.axis_index("s") * nc + lax.axis_index("c")
    base = wid * c
    pltpu.sync_copy(idx_hbm.at[pl.ds(base, c)], idx_v)
    pltpu.sync_copy(msk_hbm.at[pl.ds(base, c)], msk_v)
    iota = lax.iota(jnp.int32, L)

    def comp_body(j, carry):
        om, ou = carry
        vm32 = msk_v[pl.ds(j * L, L)]
        vm = vm32 > 0
        vidx = idx_v[pl.ds(j * L, L)]
        inc = plsc.cumsum(vm32)
        incu = plsc.cumsum(1 - vm32)
        pos_m = inc + (om - 1)
        pos_u = (ou + 1) - incu
        rows = iota + (base + j * L)
        plsc.store_scatter(perm_v, [pos_m], rows, mask=vm)
        plsc.store_scatter(pev_v, [pos_m], vidx, mask=vm)
        plsc.store_scatter(perm_v, [pos_u], rows, mask=jnp.logical_not(vm))
        cm = jnp.sum(vm32)
        return om + cm, ou - (L - cm)

    nm, _ = lax.fori_loop(0, ng, comp_body,
                          (jnp.int32(0), jnp.int32(c - 1)))

    def gather_desc(b, src_ref):
        return pltpu.make_async_copy(src_ref.at[ssts[b]], bufs[b], sems[b])

    def scatter_desc(b):
        return pltpu.make_async_copy(bufs[b], out_hbm.at[tsts[b]], sems[b])

    def run_chunks(nfull, src_ref, stage_fn):
        nsup = (nfull + NBUF - 1) // NBUF

        def sup_body(g, _):
            t0 = g * NBUF
            for b in range(NBUF):
                @pl.when(t0 + b < nfull)
                def _():
                    stage_fn(t0 + b, b)
                    gather_desc(b, src_ref).start()
            for b in range(NBUF):
                @pl.when(t0 + b < nfull)
                def _():
                    gather_desc(b, src_ref).wait()
                    scatter_desc(b).start()
            for b in range(NBUF):
                @pl.when(t0 + b < nfull)
                def _():
                    scatter_desc(b).wait()
            return 0

        lax.fori_loop(0, nsup, sup_body, 0)

    def stage_m(t, b):
        ssts[b][...] = pev_v[pl.ds(t * L, L)]
        tsts[b][...] = perm_v[pl.ds(t * L, L)]

    nfull_m = nm // L
    run_chunks(nfull_m, pe_hbm, stage_m)

    @pl.when(nm - nfull_m * L > 0)
    def _():
        start = jnp.maximum(nm - L, 0)
        kval = nm - start
        q = iota < kval
        fill_idx = jnp.full((L,), nm - 1, jnp.int32)
        fillp = plsc.load_gather(pev_v, [fill_idx])
        fillt = plsc.load_gather(perm_v, [fill_idx])
        ssts[0][...] = jnp.where(q, pev_v[pl.ds(start, L)], fillp)
        tsts[0][...] = jnp.where(q, perm_v[pl.ds(start, L)], fillt)
        gather_desc(0, pe_hbm).start()
        gather_desc(0, pe_hbm).wait()
        scatter_desc(0).start()
        scatter_desc(0).wait()

    def stage_u(u, b):
        v = perm_v[pl.ds(nm + u * L, L)]
        ssts[b][...] = v
        tsts[b][...] = v

    nu = c - nm
    nfull_u = nu // L
    run_chunks(nfull_u, x_hbm, stage_u)

    @pl.when(nu - nfull_u * L > 0)
    def _():
        q = iota >= jnp.maximum(nm - (c - L), 0)
        fill_idx = jnp.full((L,), c - 1, jnp.int32)
        fillt = plsc.load_gather(perm_v, [fill_idx])
        v = jnp.where(q, perm_v[pl.ds(c - L, L)], fillt)
        ssts[0][...] = v
        tsts[0][...] = v
        gather_desc(0, x_hbm).start()
        gather_desc(0, x_hbm).wait()
        scatter_desc(0).start()
        scatter_desc(0).wait()


def kernel(x, local_indices, group_mask, pe):
    b, s, d = x.shape
    n = b * s
    nw = 32
    c = n // nw
    x2 = x.reshape(n, d)
    idxf = local_indices.reshape(n)
    mskf = group_mask.astype(jnp.int32).reshape(n)
    mesh = plsc.VectorSubcoreMesh(core_axis_name="c", subcore_axis_name="s")
    sc_kernel = functools.partial(
        pl.kernel,
        out_type=jax.ShapeDtypeStruct((n, d), jnp.float32),
        mesh=mesh,
        compiler_params=pltpu.CompilerParams(needs_layout_passes=False),
        scratch_types=(
            [pltpu.VMEM((c,), jnp.int32)] * 4
            + [pltpu.VMEM((L, d), jnp.float32)] * NBUF
            + [pltpu.VMEM((L,), jnp.int32)] * (2 * NBUF)
            + [pltpu.SemaphoreType.DMA] * NBUF
        ),
    )(functools.partial(_sc_body, rows_per_worker=c))
    out = sc_kernel(x2, idxf, mskf, pe)
    return out.reshape(b, s, d)


# speedup vs baseline: 1.0016x; 1.0016x over previous
import functools

import jax
import jax.numpy as jnp
from jax import lax
from jax.experimental import pallas as pl
from jax.experimental.pallas import tpu as pltpu
from jax.experimental.pallas import tpu_sc as plsc

D = 1024
GROUP = 64
L = 16
CH = 32
NBUF = 3


def _sc_body(x_hbm, idx_hbm, msk_hbm, pe_hbm, out_hbm,
             idx_v, msk_v, perm_v, pev_v,
             buf0, buf1, buf2,
             sst0, sst1, sst2,
             tst0, tst1, tst2,
             gsem0, gsem1, gsem2, ssem0, ssem1, ssem2,
             *, rows_per_worker):
    c = rows_per_worker
    ng = c // L
    bufs = [buf0, buf1, buf2]
    ssts = [sst0, sst1, sst2]
    tsts = [tst0, tst1, tst2]
    gsems = [gsem0, gsem1, gsem2]
    ssems = [ssem0, ssem1, ssem2]

    nc = 2
    wid = lax.axis_index("s") * nc + lax.axis_index("c")
    base = wid * c
    pltpu.sync_copy(idx_hbm.at[pl.ds(base, c)], idx_v)
    pltpu.sync_copy(msk_hbm.at[pl.ds(base, c)], msk_v)
    iota = lax.iota(jnp.int32, L)

    def comp_body(j, carry):
        om, ou = carry
        vm32 = msk_v[pl.ds(j * L, L)]
        vm = vm32 > 0
        vidx = idx_v[pl.ds(j * L, L)]
        inc = plsc.cumsum(vm32)
        incu = plsc.cumsum(1 - vm32)
        pos_m = inc + (om - 1)
        pos_u = (ou + 1) - incu
        rows = iota + (base + j * L)
        plsc.store_scatter(perm_v, [pos_m], rows, mask=vm)
        plsc.store_scatter(pev_v, [pos_m], vidx, mask=vm)
        plsc.store_scatter(perm_v, [pos_u], rows, mask=jnp.logical_not(vm))
        cm = jnp.sum(vm32)
        return om + cm, ou - (L - cm)

    nm, _ = lax.fori_loop(0, ng, comp_body,
                          (jnp.int32(0), jnp.int32(c - 1)))
    nu = c - nm

    def run_pipeline(nchunks, src_ref, stage_fn):
        nsup = (nchunks + NBUF - 1) // NBUF

        def gd(b):
            return pltpu.make_async_copy(src_ref.at[ssts[b]], bufs[b], gsems[b])

        def sd(b):
            return pltpu.make_async_copy(bufs[b], out_hbm.at[tsts[b]], ssems[b])

        def sup_body(g, _):
            t0 = g * NBUF
            for b in range(NBUF):
                @pl.when(jnp.logical_and(g > 0, t0 - NBUF + b < nchunks))
                def _():
                    sd(b).wait()
            for b in range(NBUF):
                @pl.when(t0 + b < nchunks)
                def _():
                    stage_fn(t0 + b, b)
                    gd(b).start()
            for b in range(NBUF):
                @pl.when(t0 + b < nchunks)
                def _():
                    gd(b).wait()
                    sd(b).start()
            return 0

        lax.fori_loop(0, nsup, sup_body, 0)
        for b in range(NBUF):
            @pl.when(jnp.logical_and(nchunks > 0,
                                     (nsup - 1) * NBUF + b < nchunks))
            def _():
                sd(b).wait()

    fill_m = jnp.full((L,), jnp.maximum(nm - 1, 0), jnp.int32)
    fillp = plsc.load_gather(pev_v, [fill_m])
    fillt_m = plsc.load_gather(perm_v, [fill_m])
    nch_m = (nm + CH - 1) // CH
    wmax_m = jnp.maximum(nm - CH, 0)

    def stage_m(t, b):
        w = jnp.minimum(t * CH, wmax_m)
        for ii in range(CH // L):
            q = iota < (nm - w - ii * L)
            ssts[b][pl.ds(ii * L, L)] = jnp.where(
                q, pev_v[pl.ds(w + ii * L, L)], fillp)
            tsts[b][pl.ds(ii * L, L)] = jnp.where(
                q, perm_v[pl.ds(w + ii * L, L)], fillt_m)

    run_pipeline(nch_m, pe_hbm, stage_m)

    fill_u = jnp.full((L,), c - 1, jnp.int32)
    fillt_u = plsc.load_gather(perm_v, [fill_u])
    nch_u = (nu + CH - 1) // CH

    def stage_u(t, b):
        w = jnp.minimum(nm + t * CH, c - CH)
        for ii in range(CH // L):
            q = iota >= (nm - w - ii * L)
            v = jnp.where(q, perm_v[pl.ds(w + ii * L, L)], fillt_u)
            ssts[b][pl.ds(ii * L, L)] = v
            tsts[b][pl.ds(ii * L, L)] = v

    run_pipeline(nch_u, x_hbm, stage_u)


def kernel(x, local_indices, group_mask, pe):
    b, s, d = x.shape
    n = b * s
    nw = 32
    c = n // nw
    x2 = x.reshape(n, d)
    idxf = local_indices.reshape(n)
    mskf = group_mask.astype(jnp.int32).reshape(n)
    mesh = plsc.VectorSubcoreMesh(core_axis_name="c", subcore_axis_name="s")
    sc_kernel = functools.partial(
        pl.kernel,
        out_type=jax.ShapeDtypeStruct((n, d), jnp.float32),
        mesh=mesh,
        compiler_params=pltpu.CompilerParams(needs_layout_passes=False),
        scratch_types=(
            [pltpu.VMEM((c,), jnp.int32)] * 4
            + [pltpu.VMEM((CH, d), jnp.float32)] * NBUF
            + [pltpu.VMEM((CH,), jnp.int32)] * (2 * NBUF)
            + [pltpu.SemaphoreType.DMA] * (2 * NBUF)
        ),
    )(functools.partial(_sc_body, rows_per_worker=c))
    out = sc_kernel(x2, idxf, mskf, pe)
    return out.reshape(b, s, d)


# speedup vs baseline: 1.0282x; 1.0266x over previous
import functools

import jax
import jax.numpy as jnp
from jax import lax
from jax.experimental import pallas as pl
from jax.experimental.pallas import tpu as pltpu
from jax.experimental.pallas import tpu_sc as plsc

D = 1024
GROUP = 64
L = 16
CH = 16
NBUF = 3
NSLOT = 2 * NBUF


def _sc_body(x_hbm, idx_hbm, msk_hbm, pe_hbm, out_hbm,
             idx_v, msk_v, perm_v, pev_v,
             *scratch, rows_per_worker):
    c = rows_per_worker
    ng = c // L
    bufs = list(scratch[0:NSLOT])
    ssts = list(scratch[NSLOT:2 * NSLOT])
    tsts = list(scratch[2 * NSLOT:3 * NSLOT])
    gsems = list(scratch[3 * NSLOT:4 * NSLOT])
    ssems = list(scratch[4 * NSLOT:5 * NSLOT])

    nc = 2
    wid = lax.axis_index("s") * nc + lax.axis_index("c")
    base = wid * c
    pltpu.sync_copy(idx_hbm.at[pl.ds(base, c)], idx_v)
    pltpu.sync_copy(msk_hbm.at[pl.ds(base, c)], msk_v)
    iota = lax.iota(jnp.int32, L)

    def comp_body(j, carry):
        om, ou = carry
        vm32 = msk_v[pl.ds(j * L, L)]
        vm = vm32 > 0
        vidx = idx_v[pl.ds(j * L, L)]
        inc = plsc.cumsum(vm32)
        incu = plsc.cumsum(1 - vm32)
        pos_m = inc + (om - 1)
        pos_u = (ou + 1) - incu
        rows = iota + (base + j * L)
        plsc.store_scatter(perm_v, [pos_m], rows, mask=vm)
        plsc.store_scatter(pev_v, [pos_m], vidx, mask=vm)
        plsc.store_scatter(perm_v, [pos_u], rows, mask=jnp.logical_not(vm))
        cm = jnp.sum(vm32)
        return om + cm, ou - (L - cm)

    nm, _ = lax.fori_loop(0, ng, comp_body,
                          (jnp.int32(0), jnp.int32(c - 1)))
    nu = c - nm

    def run_pipeline(nchunks, src_ref, stage_fn):
        def gd(s):
            return pltpu.make_async_copy(src_ref.at[ssts[s]], bufs[s], gsems[s])

        def sd(s):
            return pltpu.make_async_copy(bufs[s], out_hbm.at[tsts[s]], ssems[s])

        nsup = (nchunks + NSLOT - 1) // NSLOT + 1

        def sup_body(h, _):
            for bank in range(2):
                for b in range(NBUF):
                    slot = bank * NBUF + b
                    t = (2 * h + bank) * NBUF + b
                    @pl.when(jnp.logical_and(t >= NSLOT, t - NSLOT < nchunks))
                    def _():
                        sd(slot).wait()
                    @pl.when(t < nchunks)
                    def _():
                        stage_fn(t, slot)
                        gd(slot).start()
                for b in range(NBUF):
                    slot = bank * NBUF + b
                    t = (2 * h + bank) * NBUF + b
                    @pl.when(t < nchunks)
                    def _():
                        gd(slot).wait()
                        sd(slot).start()
            return 0

        lax.fori_loop(0, nsup, sup_body, 0)

    fill_m = jnp.full((L,), jnp.maximum(nm - 1, 0), jnp.int32)
    fillp = plsc.load_gather(pev_v, [fill_m])
    fillt_m = plsc.load_gather(perm_v, [fill_m])
    nch_m = (nm + CH - 1) // CH
    wmax_m = jnp.maximum(nm - CH, 0)

    def stage_m(t, s):
        w = jnp.minimum(t * CH, wmax_m)
        for ii in range(CH // L):
            q = iota < (nm - w - ii * L)
            ssts[s][pl.ds(ii * L, L)] = jnp.where(
                q, pev_v[pl.ds(w + ii * L, L)], fillp)
            tsts[s][pl.ds(ii * L, L)] = jnp.where(
                q, perm_v[pl.ds(w + ii * L, L)], fillt_m)

    run_pipeline(nch_m, pe_hbm, stage_m)

    fill_u = jnp.full((L,), c - 1, jnp.int32)
    fillt_u = plsc.load_gather(perm_v, [fill_u])
    nch_u = (nu + CH - 1) // CH

    def stage_u(t, s):
        w = jnp.minimum(nm + t * CH, c - CH)
        for ii in range(CH // L):
            q = iota >= (nm - w - ii * L)
            v = jnp.where(q, perm_v[pl.ds(w + ii * L, L)], fillt_u)
            ssts[s][pl.ds(ii * L, L)] = v
            tsts[s][pl.ds(ii * L, L)] = v

    run_pipeline(nch_u, x_hbm, stage_u)


def kernel(x, local_indices, group_mask, pe):
    b, s, d = x.shape
    n = b * s
    nw = 32
    c = n // nw
    x2 = x.reshape(n, d)
    idxf = local_indices.reshape(n)
    mskf = group_mask.astype(jnp.int32).reshape(n)
    mesh = plsc.VectorSubcoreMesh(core_axis_name="c", subcore_axis_name="s")
    sc_kernel = functools.partial(
        pl.kernel,
        out_type=jax.ShapeDtypeStruct((n, d), jnp.float32),
        mesh=mesh,
        compiler_params=pltpu.CompilerParams(needs_layout_passes=False),
        scratch_types=(
            [pltpu.VMEM((c,), jnp.int32)] * 4
            + [pltpu.VMEM((CH, d), jnp.float32)] * NSLOT
            + [pltpu.VMEM((CH,), jnp.int32)] * (2 * NSLOT)
            + [pltpu.SemaphoreType.DMA] * (2 * NSLOT)
        ),
    )(functools.partial(_sc_body, rows_per_worker=c))
    out = sc_kernel(x2, idxf, mskf, pe)
    return out.reshape(b, s, d)
